# baseline (speedup 1.0000x reference)
import functools

import jax
import jax.numpy as jnp
from jax import lax
from jax.experimental import pallas as pl
from jax.experimental.pallas import tpu as pltpu
from jax.experimental.pallas import tpu_sc as plsc

VOCAB = 1000000
EMB = 64
B = 16384
C = 20
NC, NS = 2, 16
NW = NC * NS
BPW = B // NW
CHUNK = 128
KPB = BPW // CHUNK
NCHUNK = KPB * (1 + C)
NBUF = 4
NKB = B // CHUNK


def _gather_all(table, center2d, ctx3d):
    mesh = plsc.VectorSubcoreMesh(core_axis_name="c", subcore_axis_name="s",
                                  num_cores=NC, num_subcores=NS)

    @functools.partial(
        pl.kernel,
        out_type=(
            jax.ShapeDtypeStruct((8, 8, B), jnp.float32),
            jax.ShapeDtypeStruct((C, 8, NKB, 8, CHUNK), jnp.float32),
        ),
        mesh=mesh,
        scratch_types=[
            pltpu.VMEM((NCHUNK, CHUNK), jnp.int32),
            pltpu.VMEM((NBUF, CHUNK, EMB), jnp.float32),
            pltpu.VMEM((NBUF, 8, 8, CHUNK), jnp.float32),
        ] + [pltpu.SemaphoreType.DMA] * (2 * NBUF),
        compiler_params=pltpu.CompilerParams(use_tc_tiling_on_sc=False,
                                             needs_layout_passes=False),
    )
    def k(table_hbm, cen_hbm, ctx_hbm, outc_hbm, outx_hbm,
          idx_v, buf, tbuf, *sems):
        gsem, wsem = sems[:NBUF], sems[NBUF:]
        wid = lax.axis_index("s") * NC + lax.axis_index("c")
        kb0 = wid * KPB

        pltpu.sync_copy(cen_hbm.at[wid], idx_v.at[pl.ds(0, KPB)])
        for c in range(C):
            pltpu.sync_copy(ctx_hbm.at[c, wid],
                            idx_v.at[pl.ds(KPB * (1 + c), KPB)])

        lanes = lax.iota(jnp.int32, 16)
        rows_g = [lanes + g * 16 for g in range(8)]

        def issue_gather(j, b):
            pltpu.async_copy(table_hbm.at[idx_v.at[j]], buf.at[b], gsem[b])

        def wait_gather(j, b):
            pltpu.make_async_copy(table_hbm.at[idx_v.at[j]], buf.at[b],
                                  gsem[b]).wait()

        def transpose_chunk(b):
            src = buf.at[b]
            dst = tbuf.at[b]

            @pl.loop(0, EMB)
            def _(e):
                e_vec = lanes * 0 + e
                eb = e // 8
                em = lax.rem(e, 8)
                for g in range(8):
                    vals = plsc.load_gather(src, [rows_g[g], e_vec])
                    dst[eb, em, pl.ds(g * 16, 16)] = vals

        def issue_wb(j, b):
            @pl.when(j < KPB)
            def _():
                pltpu.async_copy(tbuf.at[b],
                                 outc_hbm.at[:, :, pl.ds((kb0 + j) * CHUNK,
                                                         CHUNK)],
                                 wsem[b])

            @pl.when(j >= KPB)
            def _():
                c = (j - KPB) // KPB
                kk = lax.rem(j - KPB, KPB)
                pltpu.async_copy(tbuf.at[b], outx_hbm.at[c, :, kb0 + kk],
                                 wsem[b])

        def wait_wb(b):
            pltpu.make_async_copy(tbuf.at[b], outx_hbm.at[0, :, 0],
                                  wsem[b]).wait()

        for b in range(NBUF):
            issue_gather(b, b)

        @pl.loop(0, NCHUNK - NBUF, step=NBUF)
        def _(j0):
            for b in range(NBUF):
                j = j0 + b
                wait_gather(j, b)
                transpose_chunk(b)
                issue_wb(j, b)
                wait_wb(b)
                issue_gather(j + NBUF, b)

        for b in range(NBUF):
            j = NCHUNK - NBUF + b
            wait_gather(j, b)
            transpose_chunk(b)
            issue_wb(j, b)
            wait_wb(b)

    return k(table, center2d, ctx3d)


def kernel(center_ids, context_ids, W_center, W_context):
    center2d = center_ids.astype(jnp.int32).reshape(NW, KPB, CHUNK)
    ctx3d = context_ids.astype(jnp.int32).T.reshape(C, NW, KPB, CHUNK)
    outc, outx = _gather_all(W_center, center2d, ctx3d)
    embs_center = outc.reshape(EMB, B).T[:, :, None]
    embs_context = outx.transpose(2, 4, 0, 1, 3).reshape(B, C, EMB)
    return (embs_center, embs_context)


# speedup vs baseline: 1.0892x; 1.0892x over previous
import functools

import jax
import jax.numpy as jnp
from jax import lax
from jax.experimental import pallas as pl
from jax.experimental.pallas import tpu as pltpu
from jax.experimental.pallas import tpu_sc as plsc

VOCAB = 1000000
EMB = 64
B = 16384
C = 20
NC, NS = 2, 16
NW = NC * NS
BPW = B // NW
CHUNK = 128
KPB = BPW // CHUNK
NCHUNK = KPB * (1 + C)
NBUF = 4
NKB = B // CHUNK
TP = CHUNK * EMB


def _gather_all(table, center2d, ctx3d):
    mesh = plsc.VectorSubcoreMesh(core_axis_name="c", subcore_axis_name="s",
                                  num_cores=NC, num_subcores=NS)

    @functools.partial(
        pl.kernel,
        out_type=(
            jax.ShapeDtypeStruct((EMB * B,), jnp.float32),
            jax.ShapeDtypeStruct((C * 8 * NKB * 8 * CHUNK,), jnp.float32),
        ),
        mesh=mesh,
        scratch_types=[
            pltpu.VMEM((NCHUNK, CHUNK), jnp.int32),
            pltpu.VMEM((NBUF, CHUNK, EMB), jnp.float32),
            pltpu.VMEM((NBUF, TP), jnp.float32),
        ] + [pltpu.SemaphoreType.DMA] * (2 * NBUF),
        compiler_params=pltpu.CompilerParams(use_tc_tiling_on_sc=False,
                                             needs_layout_passes=False),
    )
    def k(table_hbm, cen_hbm, ctx_hbm, outc_hbm, outx_hbm,
          idx_v, buf, tbuf, *sems):
        gsem, wsem = sems[:NBUF], sems[NBUF:]
        wid = lax.axis_index("s") * NC + lax.axis_index("c")
        kb0 = wid * KPB

        pltpu.sync_copy(cen_hbm.at[wid], idx_v.at[pl.ds(0, KPB)])
        for c in range(C):
            pltpu.sync_copy(ctx_hbm.at[c, wid],
                            idx_v.at[pl.ds(KPB * (1 + c), KPB)])

        lanes = lax.iota(jnp.int32, 16)
        col_base = [(lanes + kk * 16) * CHUNK for kk in range(4)]

        def issue_gather(j, b):
            pltpu.async_copy(table_hbm.at[idx_v.at[j]], buf.at[b], gsem[b])

        def wait_gather(j, b):
            pltpu.make_async_copy(table_hbm.at[idx_v.at[j]], buf.at[b],
                                  gsem[b]).wait()

        def transpose_chunk(b):
            src = buf.at[b]
            dst = tbuf.at[b]

            @pl.loop(0, CHUNK, step=8)
            def _(bm0):
                for r in range(8):
                    bm = bm0 + r
                    for kk in range(4):
                        vals = src[bm, pl.ds(kk * 16, 16)]
                        plsc.store_scatter(dst, [col_base[kk] + bm], vals)

        def issue_wb(j, b):
            @pl.when(j < KPB)
            def _():
                base = (kb0 + j) * CHUNK
                for e in range(EMB):
                    pltpu.async_copy(tbuf.at[b, pl.ds(e * CHUNK, CHUNK)],
                                     outc_hbm.at[pl.ds(e * B + base, CHUNK)],
                                     wsem[b])

            @pl.when(j >= KPB)
            def _():
                c = (j - KPB) // KPB
                kk = lax.rem(j - KPB, KPB)
                base = (c * 8 * NKB + kb0 + kk) * 1024
                for eb in range(8):
                    pltpu.async_copy(
                        tbuf.at[b, pl.ds(eb * 1024, 1024)],
                        outx_hbm.at[pl.ds(base + eb * NKB * 1024, 1024)],
                        wsem[b])

        def wait_wb(b):
            pltpu.make_async_copy(tbuf.at[b], outx_hbm.at[pl.ds(0, TP)],
                                  wsem[b]).wait()

        for b in range(NBUF):
            issue_gather(b, b)

        @pl.loop(0, NCHUNK - NBUF, step=NBUF)
        def _(j0):
            for b in range(NBUF):
                j = j0 + b
                wait_gather(j, b)
                transpose_chunk(b)
                issue_wb(j, b)
                wait_wb(b)
                issue_gather(j + NBUF, b)

        for b in range(NBUF):
            j = NCHUNK - NBUF + b
            wait_gather(j, b)
            transpose_chunk(b)
            issue_wb(j, b)
            wait_wb(b)

    return k(table, center2d, ctx3d)


def kernel(center_ids, context_ids, W_center, W_context):
    center2d = center_ids.astype(jnp.int32).reshape(NW, KPB, CHUNK)
    ctx3d = context_ids.astype(jnp.int32).T.reshape(C, NW, KPB, CHUNK)
    outc, outx = _gather_all(W_center, center2d, ctx3d)
    embs_center = outc.reshape(EMB, B).T[:, :, None]
    embs_context = (outx.reshape(C, 8, NKB, 8, CHUNK)
                    .transpose(2, 4, 0, 1, 3).reshape(B, C, EMB))
    return (embs_center, embs_context)


# speedup vs baseline: 1.4236x; 1.3070x over previous
import functools

import jax
import jax.numpy as jnp
from jax import lax
from jax.experimental import pallas as pl
from jax.experimental.pallas import tpu as pltpu
from jax.experimental.pallas import tpu_sc as plsc

VOCAB = 1000000
EMB = 64
B = 16384
C = 20
NC, NS = 2, 16
NW = NC * NS
BPW = B // NW
CHUNK = 128
KPB = BPW // CHUNK
NCHUNK = KPB * (1 + C)
NBUF = 4
NKB = B // CHUNK
PITCH = CHUNK + 1


def _gather_all(table, cen1, ctx2):
    mesh = plsc.VectorSubcoreMesh(core_axis_name="c", subcore_axis_name="s",
                                  num_cores=NC, num_subcores=NS)

    @functools.partial(
        pl.kernel,
        out_type=(
            jax.ShapeDtypeStruct((EMB * B,), jnp.float32),
            jax.ShapeDtypeStruct((C * 8 * NKB * 8 * CHUNK,), jnp.float32),
        ),
        mesh=mesh,
        scratch_types=[
            pltpu.VMEM((BPW, C), jnp.int32),
            pltpu.VMEM((BPW,), jnp.int32),
            pltpu.VMEM((NBUF, CHUNK), jnp.int32),
            pltpu.VMEM((NBUF, CHUNK, EMB), jnp.float32),
            pltpu.VMEM((NBUF, EMB, PITCH), jnp.float32),
            pltpu.VMEM((EMB * CHUNK,), jnp.float32),
        ] + [pltpu.SemaphoreType.DMA] * (2 * NBUF),
        compiler_params=pltpu.CompilerParams(use_tc_tiling_on_sc=False,
                                             needs_layout_passes=False),
    )
    def k(table_hbm, cen_hbm, ctx_hbm, outc_hbm, outx_hbm,
          ids_v, cen_v, idxrow, buf, tbuf, drain_v, *sems):
        gsem, wsem = sems[:NBUF], sems[NBUF:]
        wid = lax.axis_index("s") * NC + lax.axis_index("c")
        b0 = wid * BPW
        kb0 = wid * KPB

        pltpu.sync_copy(ctx_hbm.at[pl.ds(b0, BPW)], ids_v)
        pltpu.sync_copy(cen_hbm.at[pl.ds(b0, BPW)], cen_v)

        lanes = lax.iota(jnp.int32, 16)
        rows_g = [lanes + g * 16 for g in range(8)]
        e_vecs = [lanes + kk * 16 for kk in range(4)]

        def build_ctx_idx(c, kk, b):
            cvec = lanes * 0 + c
            for g in range(8):
                v = plsc.load_gather(ids_v, [kk * CHUNK + rows_g[g], cvec])
                idxrow[b, pl.ds(g * 16, 16)] = v

        def issue_gather(b):
            pltpu.async_copy(table_hbm.at[idxrow.at[b]], buf.at[b], gsem[b])

        def wait_gather(b):
            pltpu.make_async_copy(table_hbm.at[idxrow.at[b]], buf.at[b],
                                  gsem[b]).wait()

        def transpose_chunk(b):
            src = buf.at[b]
            dst = tbuf.at[b]

            @pl.loop(0, CHUNK, step=8)
            def _(bm0):
                for r in range(8):
                    bm = bm0 + r
                    bm_vec = lanes * 0 + bm
                    for kk in range(4):
                        vals = src[bm, pl.ds(kk * 16, 16)]
                        plsc.store_scatter(dst, [e_vecs[kk], bm_vec], vals)

        def issue_wb_runs(b, off_of_e):
            for e in range(EMB):
                pltpu.async_copy(tbuf.at[b, e, pl.ds(0, CHUNK)],
                                 off_of_e(e), wsem[b])

        def wait_wb(b):
            pltpu.make_async_copy(drain_v,
                                  outx_hbm.at[pl.ds(0, EMB * CHUNK)],
                                  wsem[b]).wait()

        @pl.loop(0, KPB)
        def _(j):
            for g in range(8):
                idxrow[0, pl.ds(g * 16, 16)] = cen_v[pl.ds(j * CHUNK + g * 16,
                                                           16)]
            issue_gather(0)
            wait_gather(0)
            transpose_chunk(0)
            base = b0 + j * CHUNK
            issue_wb_runs(0, lambda e: outc_hbm.at[pl.ds(e * B + base, CHUNK)])
            wait_wb(0)

        def ctx_c_kk(j):
            return j // KPB, lax.rem(j, KPB)

        for b in range(NBUF):
            c, kk = ctx_c_kk(b)
            build_ctx_idx(c, kk, b)
            issue_gather(b)

        def step(j, b, refill):
            wait_gather(b)
            transpose_chunk(b)
            c, kk = ctx_c_kk(j)
            base = c * (8 * NKB * 1024) + (kb0 + kk) * 1024
            issue_wb_runs(
                b, lambda e: outx_hbm.at[pl.ds(
                    base + (e // 8) * (NKB * 1024) + (e % 8) * CHUNK, CHUNK)])
            wait_wb(b)
            if refill:
                c2, kk2 = ctx_c_kk(j + NBUF)
                build_ctx_idx(c2, kk2, b)
                issue_gather(b)

        NCTX = C * KPB

        @pl.loop(0, NCTX - NBUF, step=NBUF)
        def _(j0):
            for b in range(NBUF):
                step(j0 + b, b, True)

        for b in range(NBUF):
            step(NCTX - NBUF + b, b, False)

    return k(table, cen1, ctx2)


def kernel(center_ids, context_ids, W_center, W_context):
    outc, outx = _gather_all(W_center, center_ids.astype(jnp.int32),
                             context_ids.astype(jnp.int32))
    embs_center = outc.reshape(EMB, B).T[:, :, None]
    embs_context = (outx.reshape(C, 8, NKB, 8, CHUNK)
                    .transpose(2, 4, 0, 1, 3).reshape(B, C, EMB))
    return (embs_center, embs_context)
